# baseline (speedup 1.0000x reference)
import jax
import jax.numpy as jnp
from jax import lax
from jax.experimental import pallas as pl
from jax.experimental.pallas import tpu as pltpu, tpu_sc as plsc

NUM_ROWS = 1024
ROW_LEN = 8193
SEG_LEN = 515
PAD_LEN = 16 * SEG_LEN
UNROLL = 5
NUM_STEPS = SEG_LEN // UNROLL
NC, NS = 2, 16
NW = NC * NS
ROWS_PER_W = NUM_ROWS // NW
TC_BLOCK_ROWS = 64


def _tc_seg_body(h_ref, out_ref):
    xv = h_ref[...]
    xv = xv * xv
    ids = lax.broadcasted_iota(jnp.int32, (ROW_LEN, 16), 0) // SEG_LEN
    cols = lax.broadcasted_iota(jnp.int32, (ROW_LEN, 16), 1)
    onehot = (ids == cols).astype(jnp.float32)
    out_ref[...] = jax.lax.dot_general(
        xv, onehot, (((1,), (0,)), ((), ())),
        preferred_element_type=jnp.float32,
    )


def _sc_body(h_hbm, s_hbm, g_hbm, out_hbm,
             h0, s0, g0, h1, s1, g1, binsbuf, outbuf, sem0, sem1):
    wid = lax.axis_index("s") * NC + lax.axis_index("c")
    base_row = wid * ROWS_PER_W
    zeros16 = jnp.zeros((16,), jnp.float32)
    iota16 = lax.broadcasted_iota(jnp.int32, (16,), 0)
    vidx0 = iota16 * SEG_LEN

    for buf in (h0, s0, h1, s1):
        for off in range(ROW_LEN - 1, PAD_LEN - 15, 16):
            buf[pl.ds(off, 16)] = zeros16

    def start_row(row, hbuf, sbuf, gbuf, sem):
        pltpu.make_async_copy(h_hbm.at[row], hbuf.at[pl.ds(0, ROW_LEN)], sem).start()
        pltpu.make_async_copy(s_hbm.at[row], sbuf.at[pl.ds(0, ROW_LEN)], sem).start()
        pltpu.make_async_copy(g_hbm.at[row], gbuf, sem).start()

    def wait_row(row, hbuf, sbuf, gbuf, sem):
        pltpu.make_async_copy(h_hbm.at[row], hbuf.at[pl.ds(0, ROW_LEN)], sem).wait()
        pltpu.make_async_copy(s_hbm.at[row], sbuf.at[pl.ds(0, ROW_LEN)], sem).wait()
        pltpu.make_async_copy(g_hbm.at[row], gbuf, sem).wait()

    def process_row(i2, hbuf, sbuf, gbuf, acc0, acc1):
        seg_sums = gbuf[...]
        seg_incl = plsc.cumsum(seg_sums)
        seg_base = seg_incl - seg_sums
        a_vec = jnp.full((16,), jnp.sum(seg_sums))
        inv_vec = jnp.full((16,), 16.0) / a_vec

        binsbuf[pl.ds(0, 16)] = zeros16

        def pass_b(_, carry):
            p_run, ut, vidx = carry
            hv = [plsc.load_gather(hbuf, [vidx + u]) for u in range(UNROLL)]
            sv = [plsc.load_gather(sbuf, [vidx + u]) for u in range(UNROLL)]
            xv = [v * v for v in hv]
            s01 = xv[0] + xv[1]
            s23 = xv[2] + xv[3]
            pre = (None, xv[0], s01, s01 + xv[2], s01 + s23)
            inc = [hv[u] * sv[u] for u in range(UNROLL)]
            for u in range(UNROLL):
                p_u = p_run if u == 0 else p_run + pre[u]
                scaled = p_u * inv_vec
                k = scaled.astype(jnp.int32)
                m = scaled > 0.0
                plsc.addupdate_scatter(binsbuf, [k], inc[u], mask=m)
            i01 = inc[0] + inc[1]
            i23 = inc[2] + inc[3]
            ut = ut + ((i01 + i23) + inc[4])
            p_run = p_run + ((s01 + s23) + xv[4])
            return p_run, ut, vidx + UNROLL

        _, ut, _ = lax.fori_loop(0, NUM_STEPS, pass_b, (seg_base, zeros16, vidx0))

        u_tot = jnp.full((16,), jnp.sum(ut))
        sv_bins = binsbuf[pl.ds(0, 16)]
        dev = sv_bins - u_tot * (1.0 / 16.0)
        css = jnp.sum(dev * dev)
        chisq_b = jnp.full((16,), css) * inv_vec * ((16.0 / 15.0) / 32.0)
        acc0 = acc0 + jnp.where((iota16 == i2) & (i2 < 16), chisq_b, zeros16)
        acc1 = acc1 + jnp.where((iota16 == (i2 - 16)) & (i2 >= 16), chisq_b, zeros16)
        return acc0, acc1

    start_row(base_row, h0, s0, g0, sem0)

    def pair_body(i, carry):
        acc0, acc1 = carry
        ra = base_row + 2 * i
        rb = ra + 1
        start_row(rb, h1, s1, g1, sem1)
        wait_row(ra, h0, s0, g0, sem0)
        acc0, acc1 = process_row(2 * i, h0, s0, g0, acc0, acc1)

        @pl.when(i < (ROWS_PER_W // 2 - 1))
        def _():
            start_row(ra + 2, h0, s0, g0, sem0)

        wait_row(rb, h1, s1, g1, sem1)
        acc0, acc1 = process_row(2 * i + 1, h1, s1, g1, acc0, acc1)
        return acc0, acc1

    acc0, acc1 = lax.fori_loop(0, ROWS_PER_W // 2, pair_body, (zeros16, zeros16))
    outbuf[pl.ds(0, 16)] = acc0
    outbuf[pl.ds(16, 16)] = acc1
    pltpu.sync_copy(outbuf, out_hbm.at[pl.ds(base_row, ROWS_PER_W)])


@jax.jit
def kernel(template, strain):
    h = template.reshape(NUM_ROWS, ROW_LEN)
    s = strain.reshape(NUM_ROWS, ROW_LEN)
    segs = pl.pallas_call(
        _tc_seg_body,
        out_shape=jax.ShapeDtypeStruct((NUM_ROWS, 16), jnp.float32),
        grid=(NUM_ROWS // TC_BLOCK_ROWS,),
        in_specs=[pl.BlockSpec((TC_BLOCK_ROWS, ROW_LEN), lambda i: (i, 0))],
        out_specs=pl.BlockSpec((TC_BLOCK_ROWS, 16), lambda i: (i, 0)),
    )(h)
    mesh = plsc.VectorSubcoreMesh(
        core_axis_name="c", subcore_axis_name="s", num_cores=NC, num_subcores=NS
    )
    out = pl.kernel(
        _sc_body,
        out_type=jax.ShapeDtypeStruct((NUM_ROWS,), jnp.float32),
        mesh=mesh,
        scratch_types=[
            pltpu.VMEM((PAD_LEN,), jnp.float32),
            pltpu.VMEM((PAD_LEN,), jnp.float32),
            pltpu.VMEM((16,), jnp.float32),
            pltpu.VMEM((PAD_LEN,), jnp.float32),
            pltpu.VMEM((PAD_LEN,), jnp.float32),
            pltpu.VMEM((16,), jnp.float32),
            pltpu.VMEM((32,), jnp.float32),
            pltpu.VMEM((ROWS_PER_W,), jnp.float32),
            pltpu.SemaphoreType.DMA,
            pltpu.SemaphoreType.DMA,
        ],
        compiler_params=pltpu.CompilerParams(
            needs_layout_passes=False, use_tc_tiling_on_sc=False
        ),
    )(h, s, segs)
    return out.reshape(512, 2)
